# Initial kernel scaffold; baseline (speedup 1.0000x reference)
import jax
import jax.numpy as jnp
from jax.experimental import pallas as pl


def kernel(track_context, artist_context, album_context, next_track, next_artist, next_album, track_table, artist_table, album_table):
    raise NotImplementedError("write your pallas kernel here")


# speedup vs baseline: 8.7957x; 8.7957x over previous
import functools

import jax
import jax.numpy as jnp
from jax import lax
from jax.experimental import pallas as pl
from jax.experimental.pallas import tpu as pltpu
from jax.experimental.pallas import tpu_sc as plsc

B = 4096
L = 200
F = 32
NC = 2
NS = 16
NW = NC * NS
BPW = B // NW
C1 = 128
C2 = L - C1
HALF = F // 2

_GATHER_DNUMS = lax.GatherDimensionNumbers(
    offset_dims=(), collapsed_slice_dims=(0,), start_index_map=(0,))


def _sc_body(tc_hbm, ac_hbm, alc_hbm, nt_hbm, na_hbm, nal_hbm,
             ttab_hbm, atab_hbm, altab_hbm, out_hbm,
             ctx0_v, ctx1_v, ctx2_v, nidx_v, rows_v,
             nr0_v, nr1_v, nr2_v, out_v, sem, nsem):
    wid = lax.axis_index("s") * NC + lax.axis_index("c")
    base = wid * BPW

    zero = jnp.zeros((HALF,), jnp.float32)
    lanes = lax.iota(jnp.int32, HALF)
    inv_l = jnp.float32(1.0 / L)

    tabs = (ttab_hbm, atab_hbm, altab_hbm)
    ctxs = (ctx0_v, ctx1_v, ctx2_v)
    nrows = (nr0_v, nr1_v, nr2_v)

    for ctx_hbm, next_hbm, tab_hbm, ctx_v, nr_v in (
            (tc_hbm, nt_hbm, ttab_hbm, ctx0_v, nr0_v),
            (ac_hbm, na_hbm, atab_hbm, ctx1_v, nr1_v),
            (alc_hbm, nal_hbm, altab_hbm, ctx2_v, nr2_v)):
        pltpu.sync_copy(ctx_hbm.at[pl.ds(base, BPW)], ctx_v)
        pltpu.sync_copy(next_hbm.at[pl.ds(base, BPW)], nidx_v)
        pltpu.async_copy(tab_hbm.at[nidx_v], nr_v, nsem).wait()

    def chunk_body(g, _):
        def batch_body(j, out_acc):
            i = g * HALF + j
            p = zero
            for t in range(3):
                cp1 = pltpu.async_copy(
                    tabs[t].at[ctxs[t].at[i, pl.ds(0, C1)]],
                    rows_v.at[pl.ds(0, C1)], sem)
                cp2 = pltpu.async_copy(
                    tabs[t].at[ctxs[t].at[i, pl.ds(C1, C2)]],
                    rows_v.at[pl.ds(C1, C2)], sem)
                cp1.wait()
                cp2.wait()

                def row_body(r, carry):
                    a0, a1 = carry
                    l0 = r * 8
                    for u in range(8):
                        a0 = a0 + rows_v[l0 + u, pl.ds(0, HALF)]
                        a1 = a1 + rows_v[l0 + u, pl.ds(HALF, HALF)]
                    return a0, a1

                a0, a1 = lax.fori_loop(0, L // 8, row_body, (zero, zero))
                p = (p + a0 * nrows[t][i, pl.ds(0, HALF)]
                     + a1 * nrows[t][i, pl.ds(HALF, HALF)])
            for sh in (8, 4, 2, 1):
                p = p + lax.gather(
                    p, (lanes ^ sh)[:, None], _GATHER_DNUMS, (1,),
                    mode=lax.GatherScatterMode.PROMISE_IN_BOUNDS)
            return jnp.where(lanes == j, p * inv_l, out_acc)

        out_acc = lax.fori_loop(0, HALF, batch_body, zero)
        out_v[pl.ds(g * HALF, HALF)] = out_acc
        return 0

    lax.fori_loop(0, BPW // HALF, chunk_body, 0)
    pltpu.sync_copy(out_v, out_hbm.at[pl.ds(base, BPW)])


_spotify_sc = functools.partial(
    pl.kernel,
    mesh=plsc.VectorSubcoreMesh(core_axis_name="c", subcore_axis_name="s"),
    out_type=jax.ShapeDtypeStruct((B,), jnp.float32),
    compiler_params=pltpu.CompilerParams(use_tc_tiling_on_sc=False),
    scratch_types=[
        pltpu.VMEM((BPW, L), jnp.int32),
        pltpu.VMEM((BPW, L), jnp.int32),
        pltpu.VMEM((BPW, L), jnp.int32),
        pltpu.VMEM((BPW,), jnp.int32),
        pltpu.VMEM((L, F), jnp.float32),
        pltpu.VMEM((BPW, F), jnp.float32),
        pltpu.VMEM((BPW, F), jnp.float32),
        pltpu.VMEM((BPW, F), jnp.float32),
        pltpu.VMEM((BPW,), jnp.float32),
        pltpu.SemaphoreType.DMA,
        pltpu.SemaphoreType.DMA,
    ],
)(_sc_body)


def kernel(track_context, artist_context, album_context,
           next_track, next_artist, next_album,
           track_table, artist_table, album_table):
    tc = track_context.astype(jnp.int32)
    ac = artist_context.astype(jnp.int32)
    alc = album_context.astype(jnp.int32)
    nt = next_track.reshape(-1).astype(jnp.int32)
    na = next_artist.reshape(-1).astype(jnp.int32)
    nal = next_album.reshape(-1).astype(jnp.int32)
    return _spotify_sc(tc, ac, alc, nt, na, nal,
                       track_table, artist_table, album_table)


# speedup vs baseline: 10.5939x; 1.2045x over previous
import functools

import jax
import jax.numpy as jnp
from jax import lax
from jax.experimental import pallas as pl
from jax.experimental.pallas import tpu as pltpu
from jax.experimental.pallas import tpu_sc as plsc

B = 4096
L = 200
F = 32
NC = 2
NS = 16
NW = NC * NS
BPW = B // NW
C1 = 128
C2 = L - C1
HALF = F // 2
NBUF = 4

_GATHER_DNUMS = lax.GatherDimensionNumbers(
    offset_dims=(), collapsed_slice_dims=(0,), start_index_map=(0,))


def _sc_body(tc_hbm, ac_hbm, alc_hbm, nt_hbm, na_hbm, nal_hbm,
             ttab_hbm, atab_hbm, altab_hbm, out_hbm,
             ctx_v, nidx_v, r0_v, r1_v, r2_v, r3_v, nrows_v, pacc_v, out_v,
             sem0, sem1, sem2, sem3, nsem):
    wid = lax.axis_index("s") * NC + lax.axis_index("c")
    base = wid * BPW

    zero = jnp.zeros((HALF,), jnp.float32)
    lanes = lax.iota(jnp.int32, HALF)
    bufs = (r0_v, r1_v, r2_v, r3_v)
    sems = (sem0, sem1, sem2, sem3)

    def zpacc(i, _):
        pacc_v[pl.ds(i * HALF, HALF)] = zero
        return 0

    lax.fori_loop(0, BPW, zpacc, 0)

    def issue(tab_hbm, b, buf, sm):
        pltpu.async_copy(tab_hbm.at[ctx_v.at[b, pl.ds(0, C1)]],
                         buf.at[pl.ds(0, C1)], sm)
        pltpu.async_copy(tab_hbm.at[ctx_v.at[b, pl.ds(C1, C2)]],
                         buf.at[pl.ds(C1, C2)], sm)

    def drain(tab_hbm, buf, sm):
        pltpu.make_async_copy(tab_hbm.at[ctx_v.at[0, pl.ds(0, C1)]],
                              buf.at[pl.ds(0, C1)], sm).wait()
        pltpu.make_async_copy(tab_hbm.at[ctx_v.at[0, pl.ds(C1, C2)]],
                              buf.at[pl.ds(C1, C2)], sm).wait()

    def accum(b, buf):
        def row_body(r, carry):
            a0, a1 = carry
            l0 = r * 8
            for u in range(8):
                a0 = a0 + buf[l0 + u, pl.ds(0, HALF)]
                a1 = a1 + buf[l0 + u, pl.ds(HALF, HALF)]
            return a0, a1

        a0, a1 = lax.fori_loop(0, L // 8, row_body, (zero, zero))
        p = (a0 * nrows_v[b, pl.ds(0, HALF)]
             + a1 * nrows_v[b, pl.ds(HALF, HALF)])
        off = b * HALF
        pacc_v[pl.ds(off, HALF)] = pacc_v[pl.ds(off, HALF)] + p

    for ctx_hbm, next_hbm, tab_hbm in ((tc_hbm, nt_hbm, ttab_hbm),
                                       (ac_hbm, na_hbm, atab_hbm),
                                       (alc_hbm, nal_hbm, altab_hbm)):
        pltpu.sync_copy(ctx_hbm.at[pl.ds(base, BPW)], ctx_v)
        pltpu.sync_copy(next_hbm.at[pl.ds(base, BPW)], nidx_v)
        pltpu.async_copy(tab_hbm.at[nidx_v], nrows_v, nsem).wait()

        for u in range(NBUF - 1):
            issue(tab_hbm, u, bufs[u], sems[u])

        def g_body(g, _):
            for u in range(NBUF):
                b = NBUF * g + u
                b_next = b + NBUF - 1

                @pl.when(b_next < BPW)
                def _():
                    issue(tab_hbm, b_next, bufs[(u + NBUF - 1) % NBUF],
                          sems[(u + NBUF - 1) % NBUF])

                drain(tab_hbm, bufs[u], sems[u])
                accum(b, bufs[u])
            return 0

        lax.fori_loop(0, BPW // NBUF, g_body, 0)

    inv_l = jnp.float32(1.0 / L)

    def out_chunk(g, _):
        def out_lane(j, out_acc):
            p = pacc_v[pl.ds((g * HALF + j) * HALF, HALF)]
            for sh in (8, 4, 2, 1):
                p = p + lax.gather(
                    p, (lanes ^ sh)[:, None], _GATHER_DNUMS, (1,),
                    mode=lax.GatherScatterMode.PROMISE_IN_BOUNDS)
            return jnp.where(lanes == j, p, out_acc)

        out_acc = lax.fori_loop(0, HALF, out_lane, zero)
        out_v[pl.ds(g * HALF, HALF)] = out_acc * inv_l
        return 0

    lax.fori_loop(0, BPW // HALF, out_chunk, 0)
    pltpu.sync_copy(out_v, out_hbm.at[pl.ds(base, BPW)])


_spotify_sc = functools.partial(
    pl.kernel,
    mesh=plsc.VectorSubcoreMesh(core_axis_name="c", subcore_axis_name="s"),
    out_type=jax.ShapeDtypeStruct((B,), jnp.float32),
    compiler_params=pltpu.CompilerParams(use_tc_tiling_on_sc=False),
    scratch_types=[
        pltpu.VMEM((BPW, L), jnp.int32),
        pltpu.VMEM((BPW,), jnp.int32),
        pltpu.VMEM((L, F), jnp.float32),
        pltpu.VMEM((L, F), jnp.float32),
        pltpu.VMEM((L, F), jnp.float32),
        pltpu.VMEM((L, F), jnp.float32),
        pltpu.VMEM((BPW, F), jnp.float32),
        pltpu.VMEM((BPW * HALF,), jnp.float32),
        pltpu.VMEM((BPW,), jnp.float32),
        pltpu.SemaphoreType.DMA,
        pltpu.SemaphoreType.DMA,
        pltpu.SemaphoreType.DMA,
        pltpu.SemaphoreType.DMA,
        pltpu.SemaphoreType.DMA,
    ],
)(_sc_body)


def kernel(track_context, artist_context, album_context,
           next_track, next_artist, next_album,
           track_table, artist_table, album_table):
    tc = track_context.astype(jnp.int32)
    ac = artist_context.astype(jnp.int32)
    alc = album_context.astype(jnp.int32)
    nt = next_track.reshape(-1).astype(jnp.int32)
    na = next_artist.reshape(-1).astype(jnp.int32)
    nal = next_album.reshape(-1).astype(jnp.int32)
    return _spotify_sc(tc, ac, alc, nt, na, nal,
                       track_table, artist_table, album_table)
